# baseline (speedup 1.0000x reference)
import functools

import jax
import jax.numpy as jnp
from jax.experimental import pallas as pl
from jax.experimental.pallas import tpu as pltpu

N_E = 64
D_M = 768
D_F = 1024
SEQ = 2048
TOPK = SEQ // N_E


def _moe_body(x_ref, choice_ref, w1_ref, w2_ref, out_ref, probs_ref):
    e = pl.program_id(0)

    @pl.when(e == 0)
    def _compute_probs():
        xt = x_ref[...]
        ch = choice_ref[...]
        logits_t = jax.lax.dot_general(
            ch, xt, (((1,), (1,)), ((), ())),
            preferred_element_type=jnp.float32)
        m = jnp.max(logits_t, axis=0, keepdims=True)
        p = jnp.exp(logits_t - m)
        probs_ref[...] = p / jnp.sum(p, axis=0, keepdims=True)

    row = probs_ref[pl.ds(e, 1), :]
    lane = jax.lax.broadcasted_iota(jnp.int32, (1, SEQ), 1)
    rowk = jax.lax.broadcasted_iota(jnp.int32, (TOPK, 1), 0)

    def topk_step(j, carry):
        r, idxs, gates = carry
        m = jnp.max(r, axis=1, keepdims=True)
        idx = jnp.min(jnp.where(r == m, lane, SEQ), axis=1,
                      keepdims=True)
        r = jnp.where(lane == idx, -1.0, r)
        idxs = jnp.where(rowk == j, idx, idxs)
        gates = jnp.where(rowk == j, m, gates)
        return r, idxs, gates

    idxs0 = jnp.zeros((TOPK, 1), jnp.int32)
    gates0 = jnp.zeros((TOPK, 1), jnp.float32)
    _, idxs, gates = jax.lax.fori_loop(0, TOPK, topk_step,
                                       (row, idxs0, gates0))

    lane_ks = jax.lax.broadcasted_iota(jnp.int32, (TOPK, SEQ), 1)
    P = (lane_ks == idxs).astype(jnp.float32)

    x_g = jax.lax.dot_general(P, x_ref[...], (((1,), (0,)), ((), ())),
                              preferred_element_type=jnp.float32)
    h = jax.lax.dot_general(x_g, w1_ref[0], (((1,), (1,)), ((), ())),
                            preferred_element_type=jnp.float32)
    h = h / (1.0 + jnp.exp(-h))
    y = jax.lax.dot_general(h, w2_ref[0], (((1,), (1,)), ((), ())),
                            preferred_element_type=jnp.float32)
    y = y * gates

    scat = jax.lax.dot_general(P, y, (((0,), (0,)), ((), ())),
                               preferred_element_type=jnp.float32)

    @pl.when(e == 0)
    def _init():
        out_ref[...] = scat

    @pl.when(e != 0)
    def _acc():
        out_ref[...] += scat


@functools.partial(jax.jit, static_argnames=("interpret",))
def kernel(x, choice, w1, w2, interpret=False):
    x2d = x[0]
    out = pl.pallas_call(
        _moe_body,
        grid=(N_E,),
        in_specs=[
            pl.BlockSpec((SEQ, D_M), lambda e: (0, 0)),
            pl.BlockSpec((N_E, D_M), lambda e: (0, 0)),
            pl.BlockSpec((1, D_F, D_M), lambda e: (e, 0, 0)),
            pl.BlockSpec((1, D_M, D_F), lambda e: (e, 0, 0)),
        ],
        out_specs=pl.BlockSpec((SEQ, D_M), lambda e: (0, 0)),
        out_shape=jax.ShapeDtypeStruct((SEQ, D_M), jnp.float32),
        scratch_shapes=[pltpu.VMEM((N_E, SEQ), jnp.float32)],
        compiler_params=pltpu.CompilerParams(
            dimension_semantics=("arbitrary",)),
        interpret=interpret,
    )(x2d, choice, w1, w2)
    return out[None]
